# baseline (speedup 1.0000x reference)
import functools

import jax
import jax.numpy as jnp
from jax import lax
from jax.experimental import pallas as pl
from jax.experimental.pallas import tpu as pltpu
from jax.experimental.pallas import tpu_sc as plsc


@functools.lru_cache(maxsize=None)
def _gather_fn(B0, Sp, D, NC, NS, CB, NB, K):
    NW = NC * NS
    b0_per_w = B0 // NW
    n_ch = b0_per_w // CB
    IC = CB * Sp
    n_grp = n_ch // NB
    mesh = plsc.VectorSubcoreMesh(core_axis_name="c", subcore_axis_name="s")

    @functools.partial(
        pl.kernel,
        mesh=mesh,
        out_type=jax.ShapeDtypeStruct((B0 * Sp, D), jnp.float32),
        scratch_types=[
            pltpu.VMEM((n_ch, IC), jnp.int32),
            pltpu.VMEM((NB, IC, D), jnp.float32),
        ]
        + [pltpu.SemaphoreType.DMA] * (2 * NB),
    )
    def k(table_hbm, idx_hbm, out_hbm, idx_v, rows_v, *sems):
        gsems, osems = sems[:NB], sems[NB:]
        wid = lax.axis_index("s") * NC + lax.axis_index("c")
        rbase = wid * b0_per_w * Sp
        pltpu.sync_copy(idx_hbm.at[wid], idx_v)

        def wait_write(b):
            pltpu.make_async_copy(
                rows_v.at[b], out_hbm.at[pl.ds(rbase, IC)], osems[b]
            ).wait()

        for b in range(K):
            pltpu.async_copy(table_hbm.at[idx_v.at[b]], rows_v.at[b], gsems[b])

        def body(jo, carry):
            for b in range(NB):
                j = jo * NB + b
                bp = (b + K) % NB

                @pl.when((j + K < n_ch) & (j + K >= NB))
                def _():
                    wait_write(bp)

                @pl.when(j + K < n_ch)
                def _():
                    pltpu.async_copy(
                        table_hbm.at[idx_v.at[j + K]], rows_v.at[bp], gsems[bp]
                    )

                pltpu.make_async_copy(
                    table_hbm.at[idx_v.at[j]], rows_v.at[b], gsems[b]
                ).wait()
                pltpu.async_copy(
                    rows_v.at[b], out_hbm.at[pl.ds(rbase + j * IC, IC)], osems[b]
                )

            return carry

        lax.fori_loop(0, n_grp, body, 0)

        for b in range(NB):
            wait_write(b)

    return k


def kernel(idx, table):
    B0, S = idx.shape
    V, D = table.shape
    info = plsc.get_sparse_core_info()
    NC, NS = info.num_cores, info.num_subcores
    NW = NC * NS
    CB, NB, K = 2, 8, 4
    Sp = -(-S // 8) * 8
    idx32 = idx.astype(jnp.int32)
    if Sp != S:
        idx32 = jnp.pad(idx32, ((0, 0), (0, Sp - S)))
    idx_w = idx32.reshape(NW, (B0 // NW) // CB, CB * Sp)
    out = _gather_fn(B0, Sp, D, NC, NS, CB, NB, K)(table, idx_w)
    return out.reshape(B0, Sp, D)[:, :S, :]


# speedup vs baseline: 4.9350x; 4.9350x over previous
import functools

import jax
import jax.numpy as jnp
from jax import lax
from jax.experimental import pallas as pl
from jax.experimental.pallas import tpu as pltpu
from jax.experimental.pallas import tpu_sc as plsc


@functools.lru_cache(maxsize=None)
def _gather_fn(B0, S, D, NC, NS, CB, NB, K):
    NW = NC * NS
    b0_per_w = B0 // NW
    n_ch = b0_per_w // CB
    IC = CB * S
    n_grp = n_ch // NB
    mesh = plsc.VectorSubcoreMesh(core_axis_name="c", subcore_axis_name="s")

    @functools.partial(
        pl.kernel,
        mesh=mesh,
        out_type=jax.ShapeDtypeStruct((B0, S, D), jnp.float32),
        scratch_types=[
            pltpu.VMEM((n_ch, IC), jnp.int32),
            pltpu.VMEM((NB, IC, D), jnp.float32),
        ]
        + [pltpu.SemaphoreType.DMA] * (2 * NB),
    )
    def k(table_hbm, idx_hbm, out_hbm, idx_v, rows_v, *sems):
        gsems, osems = sems[:NB], sems[NB:]
        wid = lax.axis_index("s") * NC + lax.axis_index("c")
        b0base = wid * b0_per_w
        pltpu.sync_copy(idx_hbm.at[wid], idx_v)

        def wait_writes(b):
            for c in range(CB):
                pltpu.make_async_copy(
                    rows_v.at[b, pl.ds(c * S, S)], out_hbm.at[b0base], osems[b]
                ).wait()

        for b in range(K):
            pltpu.async_copy(table_hbm.at[idx_v.at[b]], rows_v.at[b], gsems[b])

        def body(jo, carry):
            for b in range(NB):
                j = jo * NB + b
                bp = (b + K) % NB

                @pl.when((j + K < n_ch) & (j + K >= NB))
                def _():
                    wait_writes(bp)

                @pl.when(j + K < n_ch)
                def _():
                    pltpu.async_copy(
                        table_hbm.at[idx_v.at[j + K]], rows_v.at[bp], gsems[bp]
                    )

                pltpu.make_async_copy(
                    table_hbm.at[idx_v.at[j]], rows_v.at[b], gsems[b]
                ).wait()
                for c in range(CB):
                    pltpu.async_copy(
                        rows_v.at[b, pl.ds(c * S, S)],
                        out_hbm.at[b0base + j * CB + c],
                        osems[b],
                    )

            return carry

        lax.fori_loop(0, n_grp, body, 0)

        for b in range(NB):
            wait_writes(b)

    return k


def kernel(idx, table):
    B0, S = idx.shape
    V, D = table.shape
    info = plsc.get_sparse_core_info()
    NC, NS = info.num_cores, info.num_subcores
    NW = NC * NS
    CB, NB, K = 2, 8, 4
    SPLIT = 2
    B0s = B0 // SPLIT
    fn = _gather_fn(B0s, S, D, NC, NS, CB, NB, K)
    idx32 = idx.astype(jnp.int32)
    pieces = []
    for p in range(SPLIT):
        idx_w = idx32[p * B0s : (p + 1) * B0s].reshape(NW, (B0s // NW) // CB, CB * S)
        pieces.append(fn(table, idx_w))
    return jnp.concatenate(pieces, axis=0)


# speedup vs baseline: 7.8710x; 1.5949x over previous
import functools

import jax
import jax.numpy as jnp
from jax import lax
from jax.experimental import pallas as pl
from jax.experimental.pallas import tpu as pltpu
from jax.experimental.pallas import tpu_sc as plsc


@functools.lru_cache(maxsize=None)
def _gather_fn(B0, S, Sp, D, NC, NS, CB, NB, K):
    NW = NC * NS
    b0_per_w = B0 // NW
    n_ch = b0_per_w // CB
    IP = CB * Sp
    n_grp = n_ch // NB
    mesh = plsc.VectorSubcoreMesh(core_axis_name="c", subcore_axis_name="s")

    @functools.partial(
        pl.kernel,
        mesh=mesh,
        out_type=jax.ShapeDtypeStruct((B0, S, D), jnp.float32),
        scratch_types=[
            pltpu.VMEM((b0_per_w * Sp,), jnp.int32),
            pltpu.VMEM((NB, CB, S, D), jnp.float32),
        ]
        + [pltpu.SemaphoreType.DMA] * (2 * NB),
        compiler_params=pltpu.CompilerParams(use_tc_tiling_on_sc=True),
    )
    def k(table_hbm, idx_hbm, out_hbm, idx_v, rows_v, *sems):
        gsems, osems = sems[:NB], sems[NB:]
        wid = lax.axis_index("s") * NC + lax.axis_index("c")
        b0base = wid * b0_per_w
        pltpu.sync_copy(idx_hbm.at[pl.ds(b0base * Sp, b0_per_w * Sp)], idx_v)

        def start_gathers(j, b):
            for c in range(CB):
                pltpu.async_copy(
                    table_hbm.at[idx_v.at[pl.ds(j * IP + c * Sp, S)]],
                    rows_v.at[b, c],
                    gsems[b],
                )

        def wait_gathers(j, b):
            for c in range(CB):
                pltpu.make_async_copy(
                    table_hbm.at[idx_v.at[pl.ds(j * IP + c * Sp, S)]],
                    rows_v.at[b, c],
                    gsems[b],
                ).wait()

        def wait_write(b):
            pltpu.make_async_copy(
                rows_v.at[b], out_hbm.at[pl.ds(b0base, CB)], osems[b]
            ).wait()

        for b in range(K):
            start_gathers(b, b)

        def body(jo, carry):
            for b in range(NB):
                j = jo * NB + b
                bp = (b + K) % NB

                @pl.when((j + K < n_ch) & (j + K >= NB))
                def _():
                    wait_write(bp)

                @pl.when(j + K < n_ch)
                def _():
                    start_gathers(j + K, bp)

                wait_gathers(j, b)
                pltpu.async_copy(
                    rows_v.at[b],
                    out_hbm.at[pl.ds(b0base + j * CB, CB)],
                    osems[b],
                )

            return carry

        lax.fori_loop(0, n_grp, body, 0)

        for b in range(NB):
            wait_write(b)

    return k


def kernel(idx, table):
    B0, S = idx.shape
    V, D = table.shape
    info = plsc.get_sparse_core_info()
    NC, NS = info.num_cores, info.num_subcores
    NW = NC * NS
    CB, NB, K = 2, 8, 4
    Sp = -(-S // 8) * 8
    idx32 = idx.astype(jnp.int32)
    if Sp != S:
        idx32 = jnp.pad(idx32, ((0, 0), (0, Sp - S)))
    idx_w = idx32.reshape(B0 * Sp)
    out = _gather_fn(B0, S, Sp, D, NC, NS, CB, NB, K)(table, idx_w)
    return out


# speedup vs baseline: 13.7764x; 1.7503x over previous
import functools

import jax
import jax.numpy as jnp
from jax import lax
from jax.experimental import pallas as pl
from jax.experimental.pallas import tpu as pltpu
from jax.experimental.pallas import tpu_sc as plsc


@functools.lru_cache(maxsize=None)
def _gather_fn(B, D, NC, NS, CH, NB, K):
    NW = NC * NS
    b_per_w = B // NW
    n_ch = b_per_w // CH
    n_grp = n_ch // NB
    mesh = plsc.VectorSubcoreMesh(core_axis_name="c", subcore_axis_name="s")

    @functools.partial(
        pl.kernel,
        mesh=mesh,
        out_type=jax.ShapeDtypeStruct((B, D), jnp.float32),
        scratch_types=[
            pltpu.VMEM((b_per_w,), jnp.int32),
            pltpu.VMEM((NB, CH, D), jnp.float32),
        ]
        + [pltpu.SemaphoreType.DMA] * (2 * NB),
        compiler_params=pltpu.CompilerParams(use_tc_tiling_on_sc=True),
    )
    def k(table_hbm, idx_hbm, out_hbm, idx_v, rows_v, *sems):
        gsems, osems = sems[:NB], sems[NB:]
        wid = lax.axis_index("s") * NC + lax.axis_index("c")
        base = wid * b_per_w
        pltpu.sync_copy(idx_hbm.at[pl.ds(base, b_per_w)], idx_v)

        def start_gather(j, b):
            pltpu.async_copy(
                table_hbm.at[idx_v.at[pl.ds(j * CH, CH)]], rows_v.at[b], gsems[b]
            )

        def wait_gather(j, b):
            pltpu.make_async_copy(
                table_hbm.at[idx_v.at[pl.ds(j * CH, CH)]], rows_v.at[b], gsems[b]
            ).wait()

        def wait_write(b):
            pltpu.make_async_copy(
                rows_v.at[b], out_hbm.at[pl.ds(base, CH)], osems[b]
            ).wait()

        for b in range(K):
            start_gather(b, b)

        def body(jo, carry):
            for b in range(NB):
                j = jo * NB + b
                bp = (b + K) % NB

                @pl.when((j + K < n_ch) & (j + K >= NB))
                def _():
                    wait_write(bp)

                @pl.when(j + K < n_ch)
                def _():
                    start_gather(j + K, bp)

                wait_gather(j, b)
                pltpu.async_copy(
                    rows_v.at[b], out_hbm.at[pl.ds(base + j * CH, CH)], osems[b]
                )

            return carry

        lax.fori_loop(0, n_grp, body, 0)

        for b in range(NB):
            wait_write(b)

    return k


def kernel(idx, table):
    B0, S = idx.shape
    V, D = table.shape
    B = B0 * S
    info = plsc.get_sparse_core_info()
    NC, NS = info.num_cores, info.num_subcores
    CH, NB, K = 128, 5, 2
    idx_t = idx.astype(jnp.int32).T.reshape(B)
    out = _gather_fn(B, D, NC, NS, CH, NB, K)(table, idx_t)
    return out.reshape(S, B0, D).transpose(1, 0, 2)


# speedup vs baseline: 13.7769x; 1.0000x over previous
import functools

import jax
import jax.numpy as jnp
from jax import lax
from jax.experimental import pallas as pl
from jax.experimental.pallas import tpu as pltpu
from jax.experimental.pallas import tpu_sc as plsc


@functools.lru_cache(maxsize=None)
def _gather_fn(B, D, NC, NS, CH, NB, K):
    NW = NC * NS
    b_per_w = B // NW
    n_ch = b_per_w // CH
    n_grp = n_ch // NB
    mesh = plsc.VectorSubcoreMesh(core_axis_name="c", subcore_axis_name="s")

    @functools.partial(
        pl.kernel,
        mesh=mesh,
        out_type=jax.ShapeDtypeStruct((B, D), jnp.float32),
        scratch_types=[
            pltpu.VMEM((b_per_w,), jnp.int32),
            pltpu.VMEM((NB, CH, D), jnp.float32),
        ]
        + [pltpu.SemaphoreType.DMA] * (2 * NB),
        compiler_params=pltpu.CompilerParams(use_tc_tiling_on_sc=True),
    )
    def k(table_hbm, idx_hbm, out_hbm, idx_v, rows_v, *sems):
        gsems, osems = sems[:NB], sems[NB:]
        wid = lax.axis_index("s") * NC + lax.axis_index("c")
        base = wid * b_per_w
        pltpu.sync_copy(idx_hbm.at[pl.ds(base, b_per_w)], idx_v)

        def start_gather(j, b):
            pltpu.async_copy(
                table_hbm.at[idx_v.at[pl.ds(j * CH, CH)]], rows_v.at[b], gsems[b]
            )

        def wait_gather(j, b):
            pltpu.make_async_copy(
                table_hbm.at[idx_v.at[pl.ds(j * CH, CH)]], rows_v.at[b], gsems[b]
            ).wait()

        def wait_write(b):
            pltpu.make_async_copy(
                rows_v.at[b], out_hbm.at[pl.ds(base, CH)], osems[b]
            ).wait()

        for b in range(K):
            start_gather(b, b)

        def body(jo, carry):
            for b in range(NB):
                j = jo * NB + b
                bp = (b + K) % NB

                @pl.when((j + K < n_ch) & (j + K >= NB))
                def _():
                    wait_write(bp)

                @pl.when(j + K < n_ch)
                def _():
                    start_gather(j + K, bp)

                wait_gather(j, b)
                pltpu.async_copy(
                    rows_v.at[b], out_hbm.at[pl.ds(base + j * CH, CH)], osems[b]
                )

            return carry

        lax.fori_loop(0, n_grp, body, 0)

        for b in range(NB):
            wait_write(b)

    return k


def kernel(idx, table):
    B0, S = idx.shape
    V, D = table.shape
    B = B0 * S
    info = plsc.get_sparse_core_info()
    NC, NS = info.num_cores, info.num_subcores
    CH, NB, K = 128, 5, 3
    idx_t = idx.astype(jnp.int32).T.reshape(B)
    out = _gather_fn(B, D, NC, NS, CH, NB, K)(table, idx_t)
    return out.reshape(S, B0, D).transpose(1, 0, 2)


# speedup vs baseline: 13.8827x; 1.0077x over previous
import functools

import jax
import jax.numpy as jnp
from jax import lax
from jax.experimental import pallas as pl
from jax.experimental.pallas import tpu as pltpu
from jax.experimental.pallas import tpu_sc as plsc


@functools.lru_cache(maxsize=None)
def _gather_fn(B, D, NC, NS, CH, NB, K):
    NW = NC * NS
    b_per_w = B // NW
    n_ch = b_per_w // CH
    n_grp = n_ch // NB
    mesh = plsc.VectorSubcoreMesh(core_axis_name="c", subcore_axis_name="s")

    @functools.partial(
        pl.kernel,
        mesh=mesh,
        out_type=jax.ShapeDtypeStruct((B, D), jnp.float32),
        scratch_types=[
            pltpu.VMEM((b_per_w,), jnp.int32),
            pltpu.VMEM((NB, CH, D), jnp.float32),
        ]
        + [pltpu.SemaphoreType.DMA] * (2 * NB),
        compiler_params=pltpu.CompilerParams(use_tc_tiling_on_sc=True),
    )
    def k(table_hbm, idx_hbm, out_hbm, idx_v, rows_v, *sems):
        gsems, osems = sems[:NB], sems[NB:]
        wid = lax.axis_index("s") * NC + lax.axis_index("c")
        base = wid * b_per_w
        pltpu.sync_copy(idx_hbm.at[pl.ds(base, b_per_w)], idx_v)

        def start_gather(j, b):
            pltpu.async_copy(
                table_hbm.at[idx_v.at[pl.ds(j * CH, CH)]], rows_v.at[b], gsems[b]
            )

        def wait_gather(j, b):
            pltpu.make_async_copy(
                table_hbm.at[idx_v.at[pl.ds(j * CH, CH)]], rows_v.at[b], gsems[b]
            ).wait()

        def wait_write(b):
            pltpu.make_async_copy(
                rows_v.at[b], out_hbm.at[pl.ds(base, CH)], osems[b]
            ).wait()

        for b in range(K):
            start_gather(b, b)

        def body(jo, carry):
            for b in range(NB):
                j = jo * NB + b
                bp = (b + K) % NB

                @pl.when((j + K < n_ch) & (j + K >= NB))
                def _():
                    wait_write(bp)

                @pl.when(j + K < n_ch)
                def _():
                    start_gather(j + K, bp)

                wait_gather(j, b)
                pltpu.async_copy(
                    rows_v.at[b], out_hbm.at[pl.ds(base + j * CH, CH)], osems[b]
                )

            return carry

        lax.fori_loop(0, n_grp, body, 0)

        for b in range(NB):
            wait_write(b)

    return k


def kernel(idx, table):
    B0, S = idx.shape
    V, D = table.shape
    B = B0 * S
    info = plsc.get_sparse_core_info()
    NC, NS = info.num_cores, info.num_subcores
    CH, NB, K = 64, 10, 5
    idx_t = idx.astype(jnp.int32).T.reshape(B)
    out = _gather_fn(B, D, NC, NS, CH, NB, K)(table, idx_t)
    return out.reshape(S, B0, D).transpose(1, 0, 2)
